# baseline (speedup 1.0000x reference)
import functools

import jax
import jax.numpy as jnp
from jax import lax
from jax.experimental import pallas as pl
from jax.experimental.pallas import tpu as pltpu
from jax.experimental.pallas import tpu_sc as plsc

NC = 2
NS = 16
NW = NC * NS
G = 128
NBUF = 4


@functools.partial(jax.jit, static_argnames=("ng", "d"))
def _embed(idx2d, lut, *, ng, d):
    bpw = ng * G
    scale = jnp.sqrt(jnp.float32(d))

    mesh = plsc.VectorSubcoreMesh(
        core_axis_name="c", subcore_axis_name="s",
        num_cores=NC, num_subcores=NS)

    def body(idx_hbm, lut_hbm, out_hbm, idx_scr, ins, outs, sins, souts):
        wid = lax.axis_index("s") * NC + lax.axis_index("c")
        rbase = wid * ng
        obase = wid * bpw
        pltpu.sync_copy(idx_hbm.at[pl.ds(rbase, ng)], idx_scr)

        def start_gather(g, b):
            pltpu.async_copy(lut_hbm.at[idx_scr.at[g]], ins[b], sins[b])

        def wait_gather(g, b):
            pltpu.make_async_copy(
                lut_hbm.at[idx_scr.at[g]], ins[b], sins[b]).wait()

        def start_scatter(g, b):
            pltpu.async_copy(
                outs[b], out_hbm.at[pl.ds(obase + g * G, G)], souts[b])

        def wait_scatter(g, b):
            pltpu.make_async_copy(
                outs[b], out_hbm.at[pl.ds(obase + g * G, G)], souts[b]).wait()

        def scale_group(b):
            src, dst = ins[b], outs[b]

            @plsc.parallel_loop(0, G, unroll=4)
            def _(i):
                for j in range(d // 16):
                    sl = pl.ds(j * 16, 16)
                    dst[i, sl] = src[i, sl] * scale

        def do_group(g, b, first, last):
            wait_gather(g, b)
            if not first:
                wait_scatter(g, b)
            scale_group(b)
            if not last:
                start_gather(g + NBUF, b)
            start_scatter(g, b)

        for b in range(NBUF):
            start_gather(b, b)
        for b in range(NBUF):
            do_group(b, b, True, False)
        nr = ng // NBUF

        @pl.loop(1, nr - 1)
        def _(r):
            for b in range(NBUF):
                do_group(r * NBUF + b, b, False, False)

        for b in range(NBUF):
            do_group((nr - 1) * NBUF + b, b, False, True)
        for b in range(NBUF):
            wait_scatter((nr - 1) * NBUF + b, b)

    f32 = jnp.float32
    run = pl.kernel(
        body,
        out_type=jax.ShapeDtypeStruct((NW * bpw, d), f32),
        mesh=mesh,
        scratch_types=[
            pltpu.VMEM((ng, G), jnp.int32),
            tuple(pltpu.VMEM((G, d), f32) for _ in range(NBUF)),
            tuple(pltpu.VMEM((G, d), f32) for _ in range(NBUF)),
            tuple(pltpu.SemaphoreType.DMA for _ in range(NBUF)),
            tuple(pltpu.SemaphoreType.DMA for _ in range(NBUF)),
        ],
        compiler_params=pltpu.CompilerParams(use_tc_tiling_on_sc=False),
    )
    return run(idx2d, lut)


def kernel(x, lut):
    d = lut.shape[1]
    b = x.size
    assert b % (NW * G) == 0, (b, NW * G)
    ng = b // (NW * G)
    idx2d = x.astype(jnp.int32).reshape(NW * ng, G)
    out = _embed(idx2d, lut, ng=ng, d=d)
    return out.reshape(x.shape + (d,))
